# Initial kernel scaffold; baseline (speedup 1.0000x reference)
import jax
import jax.numpy as jnp
from jax.experimental import pallas as pl


def kernel(x, emb):
    raise NotImplementedError("write your pallas kernel here")


# speedup vs baseline: 2.4690x; 2.4690x over previous
import jax
import jax.numpy as jnp
from jax.experimental import pallas as pl


def _body(e_ref, o_ref):
    o_ref[...] = jnp.broadcast_to(e_ref[...][None, :, :], o_ref.shape)


def kernel(x, emb):
    b, s = x.shape
    max_seq_len, d = emb.shape
    assert s < max_seq_len
    blk = 256
    assert s % blk == 0
    out = pl.pallas_call(
        _body,
        grid=(s // blk,),
        in_specs=[pl.BlockSpec((blk, d), lambda i: (i, 0))],
        out_specs=pl.BlockSpec((b, blk, d), lambda i: (0, i, 0)),
        out_shape=jax.ShapeDtypeStruct((b, s, d), emb.dtype),
    )(emb)
    return out
